# baseline (speedup 1.0000x reference)
import functools

import jax
import jax.numpy as jnp
import numpy as np
from jax import lax
from jax.experimental import pallas as pl
from jax.experimental.pallas import tpu as pltpu
from jax.experimental.pallas import tpu_sc as plsc

N = 10000
E = 320000
D = 128
NW = 32
EPW = E // NW
C0 = 2000
CA = 80
CS = 80
BN = 1000
BE = 2000
INV_SQRT_D = float(1.0 / np.sqrt(D))

_mesh = plsc.VectorSubcoreMesh(core_axis_name="c", subcore_axis_name="s")


def _wid():
    return lax.axis_index("c") * 16 + lax.axis_index("s")


@functools.partial(
    pl.kernel,
    out_type=jax.ShapeDtypeStruct((2, N), jnp.float32),
    mesh=_mesh,
    scratch_types=[
        pltpu.VMEM((C0,), jnp.float32),
        pltpu.VMEM((C0,), jnp.int32),
        pltpu.VMEM_SHARED((N,), jnp.float32),
    ],
)
def _deg_kernel(w_hbm, row_hbm, zn_hbm, out_hbm, wbuf, ibuf, deg_sh):
    c = lax.axis_index("c")
    s = lax.axis_index("s")
    base = _wid() * EPW

    @pl.when(s == 0)
    def _():
        pltpu.sync_copy(zn_hbm, deg_sh)

    plsc.subcore_barrier()

    def body(i, carry):
        off = base + i * C0
        pltpu.sync_copy(row_hbm.at[pl.ds(off, C0)], ibuf)
        pltpu.sync_copy(w_hbm.at[pl.ds(off, C0)], wbuf)
        pltpu.sync_copy(wbuf, deg_sh.at[ibuf], add=True)
        return carry

    lax.fori_loop(0, EPW // C0, body, 0)
    plsc.subcore_barrier()

    @pl.when(s == 0)
    def _():
        pltpu.sync_copy(deg_sh, out_hbm.at[c])


@functools.partial(
    pl.kernel,
    out_type=(
        jax.ShapeDtypeStruct((E, D), jnp.float32),
        jax.ShapeDtypeStruct((E, D), jnp.float32),
        jax.ShapeDtypeStruct((E,), jnp.float32),
    ),
    mesh=_mesh,
    scratch_types=[
        pltpu.VMEM((EPW,), jnp.int32),
        pltpu.VMEM((EPW,), jnp.int32),
        pltpu.VMEM((CA, D), jnp.float32),
        pltpu.VMEM((CA, D), jnp.float32),
        pltpu.VMEM((CA,), jnp.float32),
        pltpu.SemaphoreType.DMA,
        pltpu.SemaphoreType.DMA,
        pltpu.SemaphoreType.DMA,
    ],
)
def _gather_kernel(kbv_hbm, qq_hbm, dinv_hbm, row_hbm, col_hbm,
                   rows_out, cols_out, dr_out,
                   rowspan, colspan, kbuf, qbuf, dbuf,
                   sem1, sem2, sem3):
    base = _wid() * EPW
    pltpu.sync_copy(row_hbm.at[pl.ds(base, EPW)], rowspan)
    pltpu.sync_copy(col_hbm.at[pl.ds(base, EPW)], colspan)

    def body(i, carry):
        off = i * CA
        isl = rowspan.at[pl.ds(off, CA)]
        cp1 = pltpu.async_copy(kbv_hbm.at[isl], kbuf, sem1)
        cp2 = pltpu.async_copy(
            qq_hbm.at[colspan.at[pl.ds(off, CA)]], qbuf, sem2)
        cp3 = pltpu.async_copy(dinv_hbm.at[isl], dbuf, sem3)
        cp1.wait()
        cp2.wait()
        cp3.wait()
        pltpu.sync_copy(kbuf, rows_out.at[pl.ds(base + off, CA)])
        pltpu.sync_copy(qbuf, cols_out.at[pl.ds(base + off, CA)])
        pltpu.sync_copy(dbuf, dr_out.at[pl.ds(base + off, CA)])
        return carry

    lax.fori_loop(0, EPW // CA, body, 0)


@functools.partial(
    pl.kernel,
    out_type=(
        jax.ShapeDtypeStruct((2, N, D), jnp.float32),
        jax.ShapeDtypeStruct((2, N), jnp.float32),
    ),
    mesh=_mesh,
    scratch_types=[
        pltpu.VMEM((EPW,), jnp.int32),
        pltpu.VMEM((EPW,), jnp.float32),
        pltpu.VMEM((CS,), jnp.int32),
        pltpu.VMEM((CS,), jnp.float32),
        pltpu.VMEM((CS, D), jnp.float32),
        pltpu.SemaphoreType.DMA,
        pltpu.VMEM_SHARED((N, D), jnp.float32),
        pltpu.VMEM_SHARED((N,), jnp.float32),
    ],
)
def _scatter_kernel(msg_hbm, p_hbm, col_hbm, znd_hbm, zn_hbm,
                    aggr_out, den_out,
                    colspan, pspan, cbuf, pbuf, msgbuf, sem1,
                    aggr_sh, den_sh):
    c = lax.axis_index("c")
    s = lax.axis_index("s")
    base = _wid() * EPW

    @pl.when(s == 0)
    def _():
        pltpu.sync_copy(znd_hbm, aggr_sh)
        pltpu.sync_copy(zn_hbm, den_sh)

    plsc.subcore_barrier()
    pltpu.sync_copy(col_hbm.at[pl.ds(base, EPW)], colspan)
    pltpu.sync_copy(p_hbm.at[pl.ds(base, EPW)], pspan)

    def body(i, carry):
        off = i * CS
        cp = pltpu.async_copy(msg_hbm.at[pl.ds(base + off, CS)], msgbuf, sem1)

        def cb(j, carry2):
            src = pl.ds(off + j * 16, 16)
            dst = pl.ds(j * 16, 16)
            cbuf[dst] = colspan[src]
            pbuf[dst] = pspan[src]
            return carry2

        lax.fori_loop(0, CS // 16, cb, 0)
        cp.wait()
        pltpu.sync_copy(msgbuf, aggr_sh.at[cbuf], add=True)
        pltpu.sync_copy(pbuf, den_sh.at[cbuf], add=True)
        return carry

    lax.fori_loop(0, EPW // CS, body, 0)
    plsc.subcore_barrier()

    @pl.when(s == 0)
    def _():
        pltpu.sync_copy(aggr_sh, aggr_out.at[c])
        pltpu.sync_copy(den_sh, den_out.at[c])


def _t1_body(x_ref, wk_ref, bk_ref, wq_ref, bq_ref, wv_ref, bv_ref,
             wke_ref, wko_ref, kbv_ref, qq_ref):
    xb = x_ref[...]
    mean = jnp.mean(xb, axis=1, keepdims=True)
    xc = xb - mean
    var = jnp.mean(xc * xc, axis=1, keepdims=True)
    xn = xc * lax.rsqrt(var + 1e-5)
    f32 = jnp.float32
    kb = jnp.dot(xn, wk_ref[...], preferred_element_type=f32) + bk_ref[...]
    q = jnp.dot(xn, wq_ref[...], preferred_element_type=f32) + bq_ref[...]
    v = jnp.dot(xn, wv_ref[...], preferred_element_type=f32) + bv_ref[...]
    dn = (((1,), (1,)), ((), ()))
    qks = lax.dot_general(q, wke_ref[...], dn, preferred_element_type=f32)
    qkc = lax.dot_general(q, wko_ref[...], dn, preferred_element_type=f32)
    bf16 = jnp.bfloat16
    kbv_ref[:, :D] = kb.astype(bf16)
    kbv_ref[:, D:] = v.astype(bf16)
    qq_ref[:, :D] = q.astype(bf16)
    qq_ref[:, D:D + 64] = qks.astype(bf16)
    qq_ref[:, D + 64:] = qkc.astype(bf16)


def _t2_body(p_ref, o_ref):
    sall = p_ref[0] + p_ref[1]
    o_ref[...] = jnp.where(sall == 0.0, 0.0, 1.0 / sall)


_DIV_HALF = np.array(
    [200.0 / np.power(10000.0, k / 64.0) for k in range(64)],
    dtype=np.float32).reshape(1, 64)


def _t3_body(rows_ref, cols_ref, et_ref, dr_ref, w_ref, wve_ref, wvo_ref,
             div_ref, msg_ref, p_ref):
    rows = rows_ref[...].astype(jnp.float32)
    cols = cols_ref[...].astype(jnp.float32)
    kb = rows[:, :D]
    v = rows[:, D:]
    q = cols[:, :D]
    qks = cols[:, D:D + 64]
    qkc = cols[:, D + 64:]
    ang = et_ref[...] * div_ref[...]
    te_s = jnp.sin(ang)
    te_c = jnp.cos(ang)
    att = (jnp.sum(kb * q, axis=1, keepdims=True)
           + jnp.sum(te_s * qks, axis=1, keepdims=True)
           + jnp.sum(te_c * qkc, axis=1, keepdims=True))
    att = att * jnp.float32(INV_SQRT_D) * (dr_ref[...] * w_ref[...])
    p = jnp.exp(att)
    f32 = jnp.float32
    twv = (jnp.dot(te_s, wve_ref[...], preferred_element_type=f32)
           + jnp.dot(te_c, wvo_ref[...], preferred_element_type=f32))
    msg_ref[...] = p * (v + twv)
    p_ref[...] = p


def _t4_body(x_ref, a_ref, d_ref, o_ref):
    a = a_ref[0] + a_ref[1]
    den = d_ref[0] + d_ref[1] + 1e-16
    aggr = a / den
    g = 0.5 * aggr * (1.0 + lax.erf(aggr * np.float32(0.7071067811865476)))
    o_ref[...] = x_ref[...] + g


def kernel(x, edge_index, edge_weight, x_time, edge_time,
           W_k, b_k, W_q, b_q, W_v, b_v):
    f32 = jnp.float32
    row32 = edge_index[0].astype(jnp.int32)
    col32 = edge_index[1].astype(jnp.int32)
    ew32 = edge_weight.astype(f32)
    zn = jnp.zeros((N,), f32)
    znd = jnp.zeros((N, D), f32)

    degp = _deg_kernel(ew32, row32, zn)
    dinv = pl.pallas_call(
        _t2_body,
        out_shape=jax.ShapeDtypeStruct((8, 1250), f32),
    )(degp.reshape(2, 8, 1250)).reshape(N)

    b2 = lambda b: b.reshape(1, D)
    grid_n = (N // BN,)
    kbv, qq = pl.pallas_call(
        _t1_body,
        grid=grid_n,
        in_specs=[
            pl.BlockSpec((BN, D), lambda i: (i, 0)),
            pl.BlockSpec((D, D), lambda i: (0, 0)),
            pl.BlockSpec((1, D), lambda i: (0, 0)),
            pl.BlockSpec((D, D), lambda i: (0, 0)),
            pl.BlockSpec((1, D), lambda i: (0, 0)),
            pl.BlockSpec((D, D), lambda i: (0, 0)),
            pl.BlockSpec((1, D), lambda i: (0, 0)),
            pl.BlockSpec((64, D), lambda i: (0, 0)),
            pl.BlockSpec((64, D), lambda i: (0, 0)),
        ],
        out_specs=[
            pl.BlockSpec((BN, 256), lambda i: (i, 0)),
            pl.BlockSpec((BN, 256), lambda i: (i, 0)),
        ],
        out_shape=[
            jax.ShapeDtypeStruct((N, 256), jnp.bfloat16),
            jax.ShapeDtypeStruct((N, 256), jnp.bfloat16),
        ],
    )(x, W_k, b2(b_k), W_q, b2(b_q), W_v, b2(b_v), W_k[0::2], W_k[1::2])

    kbv32 = lax.bitcast_convert_type(kbv.reshape(N, D, 2), f32)
    qq32 = lax.bitcast_convert_type(qq.reshape(N, D, 2), f32)
    rows32, cols32, dinvrow = _gather_kernel(kbv32, qq32, dinv, row32, col32)
    rows = lax.bitcast_convert_type(rows32, jnp.bfloat16).reshape(E, 256)
    cols = lax.bitcast_convert_type(cols32, jnp.bfloat16).reshape(E, 256)

    grid_e = (E // BE,)
    msg, attexp = pl.pallas_call(
        _t3_body,
        grid=grid_e,
        in_specs=[
            pl.BlockSpec((BE, 256), lambda i: (i, 0)),
            pl.BlockSpec((BE, 256), lambda i: (i, 0)),
            pl.BlockSpec((BE, 1), lambda i: (i, 0)),
            pl.BlockSpec((BE, 1), lambda i: (i, 0)),
            pl.BlockSpec((BE, 1), lambda i: (i, 0)),
            pl.BlockSpec((64, D), lambda i: (0, 0)),
            pl.BlockSpec((64, D), lambda i: (0, 0)),
            pl.BlockSpec((1, 64), lambda i: (0, 0)),
        ],
        out_specs=[
            pl.BlockSpec((BE, D), lambda i: (i, 0)),
            pl.BlockSpec((BE, 1), lambda i: (i, 0)),
        ],
        out_shape=[
            jax.ShapeDtypeStruct((E, D), f32),
            jax.ShapeDtypeStruct((E, 1), f32),
        ],
    )(rows, cols, edge_time.reshape(E, 1), dinvrow.reshape(E, 1),
      ew32.reshape(E, 1), W_v[0::2], W_v[1::2], jnp.asarray(_DIV_HALF))

    aggrp, denp = _scatter_kernel(msg, attexp.reshape(E), col32, znd, zn)

    out = pl.pallas_call(
        _t4_body,
        grid=grid_n,
        in_specs=[
            pl.BlockSpec((BN, D), lambda i: (i, 0)),
            pl.BlockSpec((2, BN, D), lambda i: (0, i, 0)),
            pl.BlockSpec((2, BN, 1), lambda i: (0, i, 0)),
        ],
        out_specs=pl.BlockSpec((BN, D), lambda i: (i, 0)),
        out_shape=jax.ShapeDtypeStruct((N, D), f32),
    )(x, aggrp, denp.reshape(2, N, 1))
    return out


# speedup vs baseline: 4.0709x; 4.0709x over previous
import functools

import jax
import jax.numpy as jnp
import numpy as np
from jax import lax
from jax.experimental import pallas as pl
from jax.experimental.pallas import tpu as pltpu
from jax.experimental.pallas import tpu_sc as plsc

N = 10000
E = 320000
D = 128
NW = 32
EPW = E // NW
C0 = 2000
CA = 80
CS = 80
BN = 1000
BE = 4000
INV_SQRT_D = float(1.0 / np.sqrt(D))

_mesh = plsc.VectorSubcoreMesh(core_axis_name="c", subcore_axis_name="s")


def _wid():
    return lax.axis_index("c") * 16 + lax.axis_index("s")


@functools.partial(
    pl.kernel,
    out_type=jax.ShapeDtypeStruct((2, N), jnp.float32),
    mesh=_mesh,
    scratch_types=[
        pltpu.VMEM((C0,), jnp.float32),
        pltpu.VMEM((C0,), jnp.int32),
        pltpu.VMEM_SHARED((N,), jnp.float32),
    ],
)
def _deg_kernel(w_hbm, row_hbm, zn_hbm, out_hbm, wbuf, ibuf, deg_sh):
    c = lax.axis_index("c")
    s = lax.axis_index("s")
    base = _wid() * EPW

    @pl.when(s == 0)
    def _():
        pltpu.sync_copy(zn_hbm, deg_sh)

    plsc.subcore_barrier()

    def body(i, carry):
        off = base + i * C0
        pltpu.sync_copy(row_hbm.at[pl.ds(off, C0)], ibuf)
        pltpu.sync_copy(w_hbm.at[pl.ds(off, C0)], wbuf)
        pltpu.sync_copy(wbuf, deg_sh.at[ibuf], add=True)
        return carry

    lax.fori_loop(0, EPW // C0, body, 0)
    plsc.subcore_barrier()

    @pl.when(s == 0)
    def _():
        pltpu.sync_copy(deg_sh, out_hbm.at[c])


_NCH = EPW // CA


@functools.partial(
    pl.kernel,
    out_type=(
        jax.ShapeDtypeStruct((E, D), jnp.float32),
        jax.ShapeDtypeStruct((E, D), jnp.float32),
        jax.ShapeDtypeStruct((E,), jnp.float32),
    ),
    mesh=_mesh,
    scratch_types=[
        pltpu.VMEM((EPW,), jnp.int32),
        pltpu.VMEM((EPW,), jnp.int32),
        pltpu.VMEM((CA, D), jnp.float32),
        pltpu.VMEM((CA, D), jnp.float32),
        pltpu.VMEM((CA, D), jnp.float32),
        pltpu.VMEM((CA, D), jnp.float32),
        pltpu.VMEM((CA,), jnp.float32),
        pltpu.VMEM((CA,), jnp.float32),
        pltpu.SemaphoreType.DMA,
        pltpu.SemaphoreType.DMA,
        pltpu.SemaphoreType.DMA,
        pltpu.SemaphoreType.DMA,
        pltpu.SemaphoreType.DMA,
        pltpu.SemaphoreType.DMA,
    ],
)
def _gather_kernel(kbv_hbm, qq_hbm, dinv_hbm, row_hbm, col_hbm,
                   rows_out, cols_out, dr_out,
                   rowspan, colspan, kbuf0, kbuf1, qbuf0, qbuf1,
                   dbuf0, dbuf1, sk0, sq0, sd0, sk1, sq1, sd1):
    base = _wid() * EPW
    pltpu.sync_copy(row_hbm.at[pl.ds(base, EPW)], rowspan)
    pltpu.sync_copy(col_hbm.at[pl.ds(base, EPW)], colspan)

    def fire(off, kbuf, qbuf, dbuf, sk, sq, sd):
        isl = rowspan.at[pl.ds(off, CA)]
        pltpu.async_copy(kbv_hbm.at[isl], kbuf, sk)
        pltpu.async_copy(qq_hbm.at[colspan.at[pl.ds(off, CA)]], qbuf, sq)
        pltpu.async_copy(dinv_hbm.at[isl], dbuf, sd)

    def drain(kbuf, qbuf, dbuf, sk, sq, sd):
        pltpu.make_async_copy(kbv_hbm.at[pl.ds(0, CA)], kbuf, sk).wait()
        pltpu.make_async_copy(qq_hbm.at[pl.ds(0, CA)], qbuf, sq).wait()
        pltpu.make_async_copy(dinv_hbm.at[pl.ds(0, CA)], dbuf, sd).wait()

    def wout(off, kbuf, qbuf, dbuf):
        pltpu.sync_copy(kbuf, rows_out.at[pl.ds(base + off, CA)])
        pltpu.sync_copy(qbuf, cols_out.at[pl.ds(base + off, CA)])
        pltpu.sync_copy(dbuf, dr_out.at[pl.ds(base + off, CA)])

    fire(0, kbuf0, qbuf0, dbuf0, sk0, sq0, sd0)

    def body(t, carry):
        c0 = 2 * t
        fire((c0 + 1) * CA, kbuf1, qbuf1, dbuf1, sk1, sq1, sd1)
        drain(kbuf0, qbuf0, dbuf0, sk0, sq0, sd0)
        wout(c0 * CA, kbuf0, qbuf0, dbuf0)
        fire((c0 + 2) * CA, kbuf0, qbuf0, dbuf0, sk0, sq0, sd0)
        drain(kbuf1, qbuf1, dbuf1, sk1, sq1, sd1)
        wout((c0 + 1) * CA, kbuf1, qbuf1, dbuf1)
        return carry

    lax.fori_loop(0, (_NCH - 1) // 2, body, 0)
    drain(kbuf0, qbuf0, dbuf0, sk0, sq0, sd0)
    wout((_NCH - 1) * CA, kbuf0, qbuf0, dbuf0)


@functools.partial(
    pl.kernel,
    out_type=(
        jax.ShapeDtypeStruct((2, N, D), jnp.float32),
        jax.ShapeDtypeStruct((2, N), jnp.float32),
    ),
    mesh=_mesh,
    scratch_types=[
        pltpu.VMEM((EPW,), jnp.int32),
        pltpu.VMEM((EPW,), jnp.float32),
        pltpu.VMEM((CS,), jnp.int32),
        pltpu.VMEM((CS,), jnp.float32),
        pltpu.VMEM((CS, D), jnp.float32),
        pltpu.VMEM((CS, D), jnp.float32),
        pltpu.SemaphoreType.DMA,
        pltpu.SemaphoreType.DMA,
        pltpu.VMEM_SHARED((N, D), jnp.float32),
        pltpu.VMEM_SHARED((N,), jnp.float32),
    ],
)
def _scatter_kernel(msg_hbm, p_hbm, col_hbm, znd_hbm, zn_hbm,
                    aggr_out, den_out,
                    colspan, pspan, cbuf, pbuf, mbuf0, mbuf1, sm0, sm1,
                    aggr_sh, den_sh):
    c = lax.axis_index("c")
    s = lax.axis_index("s")
    base = _wid() * EPW

    @pl.when(s == 0)
    def _():
        pltpu.sync_copy(znd_hbm, aggr_sh)
        pltpu.sync_copy(zn_hbm, den_sh)

    plsc.subcore_barrier()
    pltpu.sync_copy(col_hbm.at[pl.ds(base, EPW)], colspan)
    pltpu.sync_copy(p_hbm.at[pl.ds(base, EPW)], pspan)

    def fire(off, mbuf, sm):
        pltpu.async_copy(msg_hbm.at[pl.ds(base + off, CS)], mbuf, sm)

    def drain(mbuf, sm):
        pltpu.make_async_copy(msg_hbm.at[pl.ds(0, CS)], mbuf, sm).wait()

    def scat(off, mbuf):
        for o in range(0, CS, 16):
            cbuf[pl.ds(o, 16)] = colspan[pl.ds(off + o, 16)]
            pbuf[pl.ds(o, 16)] = pspan[pl.ds(off + o, 16)]
        pltpu.sync_copy(mbuf, aggr_sh.at[cbuf], add=True)
        pltpu.sync_copy(pbuf, den_sh.at[cbuf], add=True)

    fire(0, mbuf0, sm0)

    def body(t, carry):
        c0 = 2 * t
        fire((c0 + 1) * CS, mbuf1, sm1)
        drain(mbuf0, sm0)
        scat(c0 * CS, mbuf0)
        fire((c0 + 2) * CS, mbuf0, sm0)
        drain(mbuf1, sm1)
        scat((c0 + 1) * CS, mbuf1)
        return carry

    lax.fori_loop(0, (EPW // CS - 1) // 2, body, 0)
    drain(mbuf0, sm0)
    scat((EPW // CS - 1) * CS, mbuf0)

    plsc.subcore_barrier()

    @pl.when(s == 0)
    def _():
        pltpu.sync_copy(aggr_sh, aggr_out.at[c])
        pltpu.sync_copy(den_sh, den_out.at[c])


def _t1_body(x_ref, wk_ref, bk_ref, wq_ref, bq_ref, wv_ref, bv_ref,
             wke_ref, wko_ref, kbv_ref, qq_ref):
    xb = x_ref[...]
    mean = jnp.mean(xb, axis=1, keepdims=True)
    xc = xb - mean
    var = jnp.mean(xc * xc, axis=1, keepdims=True)
    xn = xc * lax.rsqrt(var + 1e-5)
    f32 = jnp.float32
    kb = jnp.dot(xn, wk_ref[...], preferred_element_type=f32) + bk_ref[...]
    q = jnp.dot(xn, wq_ref[...], preferred_element_type=f32) + bq_ref[...]
    v = jnp.dot(xn, wv_ref[...], preferred_element_type=f32) + bv_ref[...]
    dn = (((1,), (1,)), ((), ()))
    qks = lax.dot_general(q, wke_ref[...], dn, preferred_element_type=f32)
    qkc = lax.dot_general(q, wko_ref[...], dn, preferred_element_type=f32)
    qk_sc = jnp.concatenate([qks, qkc], axis=1)

    def bfbits(t):
        return lax.bitcast_convert_type(
            t.astype(jnp.bfloat16).astype(f32), jnp.int32)

    def pack(lo, hi):
        w = lax.bitwise_or(bfbits(hi),
                           lax.shift_right_logical(bfbits(lo), 16))
        return lax.bitcast_convert_type(w, f32)

    kbv_ref[...] = pack(kb, v)
    qq_ref[...] = pack(q, qk_sc)


def _t2_body(p_ref, o_ref):
    sall = p_ref[0] + p_ref[1]
    o_ref[...] = jnp.where(sall == 0.0, 0.0, 1.0 / sall)


_DIV_HALF = np.array(
    [200.0 / np.power(10000.0, k / 64.0) for k in range(64)],
    dtype=np.float32).reshape(1, 64)


def _t3_body(rows_ref, cols_ref, et_ref, dr_ref, w_ref, wvsc_ref,
             div_ref, msg_ref, p_ref):
    f32 = jnp.float32
    i32 = jnp.int32
    himask = jnp.int32(-65536)

    u_r = lax.bitcast_convert_type(rows_ref[...], i32)
    kb = lax.bitcast_convert_type(lax.shift_left(u_r, 16), f32)
    v = lax.bitcast_convert_type(lax.bitwise_and(u_r, himask), f32)
    u_c = lax.bitcast_convert_type(cols_ref[...], i32)
    q = lax.bitcast_convert_type(lax.shift_left(u_c, 16), f32)
    qk_sc = lax.bitcast_convert_type(lax.bitwise_and(u_c, himask), f32)

    ang = et_ref[...] * div_ref[...]
    k = (ang * np.float32(0.15915494309189535) + np.float32(12582912.0)
         ) - np.float32(12582912.0)
    rr = ang - k * np.float32(6.283185307179586)
    r2 = rr * rr
    s = rr * (np.float32(0.99998459) + r2 * (np.float32(-0.16663258)
        + r2 * (np.float32(0.008312383) + r2 * (np.float32(-1.9316182e-4)
        + r2 * np.float32(2.1732101e-6)))))
    co = (np.float32(0.99999944) + r2 * (np.float32(-0.49999558)
        + r2 * (np.float32(0.041661032) + r2 * (np.float32(-1.3862743e-3)
        + r2 * (np.float32(2.4253138e-5) + r2 * np.float32(-2.2193694e-7))))))
    te_sc = jnp.concatenate([s, co], axis=1)

    att = (jnp.sum(kb * q, axis=1, keepdims=True)
           + jnp.sum(te_sc * qk_sc, axis=1, keepdims=True))
    att = att * jnp.float32(INV_SQRT_D) * (dr_ref[...] * w_ref[...])
    p = jnp.exp(att)
    twv = jnp.dot(te_sc, wvsc_ref[...], preferred_element_type=f32)
    msg_ref[...] = p * (v + twv)
    p_ref[...] = p


def _t4_body(x_ref, a_ref, d_ref, o_ref):
    a = a_ref[0] + a_ref[1]
    den = d_ref[0] + d_ref[1] + 1e-16
    aggr = a / den
    g = 0.5 * aggr * (1.0 + lax.erf(aggr * np.float32(0.7071067811865476)))
    o_ref[...] = x_ref[...] + g


def kernel(x, edge_index, edge_weight, x_time, edge_time,
           W_k, b_k, W_q, b_q, W_v, b_v):
    f32 = jnp.float32
    row32 = edge_index[0].astype(jnp.int32)
    col32 = edge_index[1].astype(jnp.int32)
    ew32 = edge_weight.astype(f32)
    zn = jnp.zeros((N,), f32)
    znd = jnp.zeros((N, D), f32)

    degp = _deg_kernel(ew32, row32, zn)
    dinv = pl.pallas_call(
        _t2_body,
        out_shape=jax.ShapeDtypeStruct((8, 1250), f32),
    )(degp.reshape(2, 8, 1250)).reshape(N)

    b2 = lambda b: b.reshape(1, D)
    grid_n = (N // BN,)
    kbv, qq = pl.pallas_call(
        _t1_body,
        grid=grid_n,
        in_specs=[
            pl.BlockSpec((BN, D), lambda i: (i, 0)),
            pl.BlockSpec((D, D), lambda i: (0, 0)),
            pl.BlockSpec((1, D), lambda i: (0, 0)),
            pl.BlockSpec((D, D), lambda i: (0, 0)),
            pl.BlockSpec((1, D), lambda i: (0, 0)),
            pl.BlockSpec((D, D), lambda i: (0, 0)),
            pl.BlockSpec((1, D), lambda i: (0, 0)),
            pl.BlockSpec((64, D), lambda i: (0, 0)),
            pl.BlockSpec((64, D), lambda i: (0, 0)),
        ],
        out_specs=[
            pl.BlockSpec((BN, D), lambda i: (i, 0)),
            pl.BlockSpec((BN, D), lambda i: (i, 0)),
        ],
        out_shape=[
            jax.ShapeDtypeStruct((N, D), f32),
            jax.ShapeDtypeStruct((N, D), f32),
        ],
    )(x, W_k, b2(b_k), W_q, b2(b_q), W_v, b2(b_v), W_k[0::2], W_k[1::2])

    rows32, cols32, dinvrow = _gather_kernel(kbv, qq, dinv, row32, col32)

    grid_e = (E // BE,)
    msg, attexp = pl.pallas_call(
        _t3_body,
        grid=grid_e,
        in_specs=[
            pl.BlockSpec((BE, D), lambda i: (i, 0)),
            pl.BlockSpec((BE, D), lambda i: (i, 0)),
            pl.BlockSpec((BE, 1), lambda i: (i, 0)),
            pl.BlockSpec((BE, 1), lambda i: (i, 0)),
            pl.BlockSpec((BE, 1), lambda i: (i, 0)),
            pl.BlockSpec((D, D), lambda i: (0, 0)),
            pl.BlockSpec((1, 64), lambda i: (0, 0)),
        ],
        out_specs=[
            pl.BlockSpec((BE, D), lambda i: (i, 0)),
            pl.BlockSpec((BE, 1), lambda i: (i, 0)),
        ],
        out_shape=[
            jax.ShapeDtypeStruct((E, D), f32),
            jax.ShapeDtypeStruct((E, 1), f32),
        ],
    )(rows32, cols32, edge_time.reshape(E, 1), dinvrow.reshape(E, 1),
      ew32.reshape(E, 1), jnp.concatenate([W_v[0::2], W_v[1::2]], axis=0),
      jnp.asarray(_DIV_HALF))

    aggrp, denp = _scatter_kernel(msg, attexp.reshape(E), col32, znd, zn)

    out = pl.pallas_call(
        _t4_body,
        grid=grid_n,
        in_specs=[
            pl.BlockSpec((BN, D), lambda i: (i, 0)),
            pl.BlockSpec((2, BN, D), lambda i: (0, i, 0)),
            pl.BlockSpec((2, BN, 1), lambda i: (0, i, 0)),
        ],
        out_specs=pl.BlockSpec((BN, D), lambda i: (i, 0)),
        out_shape=jax.ShapeDtypeStruct((N, D), f32),
    )(x, aggrp, denp.reshape(2, N, 1))
    return out
